# Initial kernel scaffold; baseline (speedup 1.0000x reference)
import jax
import jax.numpy as jnp
from jax.experimental import pallas as pl


def kernel(pieces, emb_table, W, b):
    raise NotImplementedError("write your pallas kernel here")


# speedup vs baseline: 2.6004x; 2.6004x over previous
import functools

import jax
import jax.numpy as jnp
from jax import lax
from jax.experimental import pallas as pl
from jax.experimental.pallas import tpu as pltpu
from jax.experimental.pallas import tpu_sc as plsc

B = 16384
S = 50
D = 64
T = 1000

NC = 2
NS = 16
NW = NC * NS
BPW = B // NW
CB = 8
NCHUNK = BPW // CB
ROWS = CB * S
NREG = D // 16


def _sc_embed_sum(pieces_flat, table):
    mesh = plsc.VectorSubcoreMesh(core_axis_name="c", subcore_axis_name="s")

    @functools.partial(
        pl.kernel,
        mesh=mesh,
        out_type=jax.ShapeDtypeStruct((B, D), jnp.float32),
        compiler_params=pltpu.CompilerParams(use_tc_tiling_on_sc=False),
        scratch_types=[
            pltpu.VMEM((BPW * S,), jnp.int32),
            pltpu.VMEM((ROWS, D), jnp.float32),
            pltpu.VMEM((ROWS, D), jnp.float32),
            pltpu.VMEM((BPW, D), jnp.float32),
            pltpu.SemaphoreType.DMA,
            pltpu.SemaphoreType.DMA,
        ],
    )
    def k(pieces_hbm, table_hbm, out_hbm, idx_v, rows0, rows1, feat_v, sem0, sem1):
        wid = lax.axis_index("s") * NC + lax.axis_index("c")
        base = wid * (BPW * S)
        pltpu.sync_copy(pieces_hbm.at[pl.ds(base, BPW * S)], idx_v)
        rows = (rows0, rows1)
        sems = (sem0, sem1)

        def gather(chunk, p):
            off = pl.multiple_of(chunk * ROWS, 8)
            return pltpu.make_async_copy(
                table_hbm.at[idx_v.at[pl.ds(off, ROWS)]], rows[p], sems[p]
            )

        for p in range(2):
            gather(p, p).start()

        def accumulate(buf, chunk):
            def per_b(bi, _):
                r0 = bi * S
                acc = [buf[r0, pl.ds(r * 16, 16)] for r in range(NREG)]

                def per_s(si, acc):
                    rr = r0 + 1 + si * 7
                    for j in range(7):
                        for r in range(NREG):
                            acc[r] = acc[r] + buf[rr + j, pl.ds(r * 16, 16)]
                    return acc

                acc = lax.fori_loop(0, 7, per_s, acc)
                row = chunk * CB + bi
                for r in range(NREG):
                    feat_v[row, pl.ds(r * 16, 16)] = acc[r]
                return 0

            lax.fori_loop(0, CB, per_b, 0)

        def outer(i, _):
            g = i * 2
            for p in range(2):
                chunk = g + p
                gather(chunk, p).wait()

                @pl.when(chunk + 2 < NCHUNK)
                def _():
                    gather(chunk + 2, p).start()

                accumulate(rows[p], chunk)
            return 0

        lax.fori_loop(0, NCHUNK // 2, outer, 0)
        out_off = pl.multiple_of(wid * BPW, 8)
        pltpu.sync_copy(feat_v, out_hbm.at[pl.ds(out_off, BPW)])

    return k(pieces_flat, table)


def _tc_head(feature, W, b2):
    BB = 1024

    def body(x_ref, w_ref, b_ref, o_ref):
        z = lax.dot_general(
            x_ref[...], w_ref[...], (((1,), (1,)), ((), ())),
            preferred_element_type=jnp.float32,
        )
        z = z + b_ref[...]
        o_ref[...] = 1.0 / (1.0 + jnp.exp(-z))

    return pl.pallas_call(
        body,
        grid=(B // BB,),
        in_specs=[
            pl.BlockSpec((BB, D), lambda i: (i, 0)),
            pl.BlockSpec((T, D), lambda i: (0, 0)),
            pl.BlockSpec((1, T), lambda i: (0, 0)),
        ],
        out_specs=pl.BlockSpec((BB, T), lambda i: (i, 0)),
        out_shape=jax.ShapeDtypeStruct((B, T), jnp.float32),
    )(feature, W, b2)


def kernel(pieces, emb_table, W, b):
    feature = _sc_embed_sum(pieces.reshape(-1), emb_table)
    return _tc_head(feature, W, b.reshape(1, T))


# speedup vs baseline: 2.8278x; 1.0874x over previous
import functools

import jax
import jax.numpy as jnp
from jax import lax
from jax.experimental import pallas as pl
from jax.experimental.pallas import tpu as pltpu
from jax.experimental.pallas import tpu_sc as plsc

B = 16384
S = 50
D = 64
T = 1000

NC = 2
NS = 16
NW = NC * NS
BPW = B // NW
CB = 8
NCHUNK = BPW // CB
ROWS = CB * S
NREG = D // 16


def _sc_embed_sum(pieces_flat, table):
    mesh = plsc.VectorSubcoreMesh(core_axis_name="c", subcore_axis_name="s")

    @functools.partial(
        pl.kernel,
        mesh=mesh,
        out_type=jax.ShapeDtypeStruct((B, D), jnp.float32),
        compiler_params=pltpu.CompilerParams(use_tc_tiling_on_sc=False),
        scratch_types=[
            pltpu.VMEM((BPW * S,), jnp.int32),
            pltpu.VMEM((ROWS, D), jnp.float32),
            pltpu.VMEM((ROWS, D), jnp.float32),
            pltpu.VMEM((BPW, D), jnp.float32),
            pltpu.SemaphoreType.DMA,
            pltpu.SemaphoreType.DMA,
        ],
    )
    def k(pieces_hbm, table_hbm, out_hbm, idx_v, rows0, rows1, feat_v, sem0, sem1):
        wid = lax.axis_index("s") * NC + lax.axis_index("c")
        base = wid * (BPW * S)
        pltpu.sync_copy(pieces_hbm.at[pl.ds(base, BPW * S)], idx_v)
        rows = (rows0, rows1)
        sems = (sem0, sem1)

        def gather(chunk, p):
            off = pl.multiple_of(chunk * ROWS, 8)
            return pltpu.make_async_copy(
                table_hbm.at[idx_v.at[pl.ds(off, ROWS)]], rows[p], sems[p]
            )

        for p in range(2):
            gather(p, p).start()

        def accumulate(buf, chunk):
            def per_b(bi, _):
                r0 = bi * S
                acc = [buf[r0, pl.ds(r * 16, 16)] for r in range(NREG)]

                def per_s(si, acc):
                    rr = r0 + 1 + si * 7
                    for j in range(7):
                        for r in range(NREG):
                            acc[r] = acc[r] + buf[rr + j, pl.ds(r * 16, 16)]
                    return acc

                acc = lax.fori_loop(0, 7, per_s, acc)
                row = chunk * CB + bi
                for r in range(NREG):
                    feat_v[row, pl.ds(r * 16, 16)] = acc[r]
                return 0

            lax.fori_loop(0, CB, per_b, 0)

        def outer(i, _):
            g = i * 2
            for p in range(2):
                chunk = g + p
                gather(chunk, p).wait()

                @pl.when(chunk + 2 < NCHUNK)
                def _():
                    gather(chunk + 2, p).start()

                accumulate(rows[p], chunk)
            return 0

        lax.fori_loop(0, NCHUNK // 2, outer, 0)
        out_off = pl.multiple_of(wid * BPW, 8)
        pltpu.sync_copy(feat_v, out_hbm.at[pl.ds(out_off, BPW)])

    return k(pieces_flat, table)


def _tc_head(feature, W, b2):
    BB = 2048

    def body(x_ref, w_ref, b_ref, o_ref):
        z = lax.dot_general(
            w_ref[...], x_ref[...], (((1,), (1,)), ((), ())),
            preferred_element_type=jnp.float32,
        )
        z = z + b_ref[...]
        o_ref[...] = 1.0 / (1.0 + jnp.exp(-z))

    return pl.pallas_call(
        body,
        grid=(B // BB,),
        in_specs=[
            pl.BlockSpec((BB, D), lambda i: (i, 0)),
            pl.BlockSpec((T, D), lambda i: (0, 0)),
            pl.BlockSpec((T, 1), lambda i: (0, 0)),
        ],
        out_specs=pl.BlockSpec((T, BB), lambda i: (0, i)),
        out_shape=jax.ShapeDtypeStruct((T, B), jnp.float32),
    )(feature, W, b2)


def kernel(pieces, emb_table, W, b):
    feature = _sc_embed_sum(pieces.reshape(-1), emb_table)
    return _tc_head(feature, W, b.reshape(T, 1)).T
